# Initial kernel scaffold; baseline (speedup 1.0000x reference)
import jax
import jax.numpy as jnp
from jax.experimental import pallas as pl


def kernel(x, edge_index, edge_attr, We1, be1, W1a, b1a, W1b, b1b, Wem, bem, Wm1, bm1, Wm2, bm2, Wel, bel, Wl1, bl1, Wl2, bl2):
    raise NotImplementedError("write your pallas kernel here")


# speedup vs baseline: 2.2187x; 2.2187x over previous
import functools
import jax
import jax.numpy as jnp
from jax import lax
from jax.experimental import pallas as pl
from jax.experimental.pallas import tpu as pltpu
from jax.experimental.pallas import tpu_sc as plsc

NC = 2
NS = 16
NW = NC * NS
G = 128


def _ceil_to(a, b):
    return (a + b - 1) // b * b


def _edense_body(ea_ref, w_ref, b_ref, e1_ref, eml_ref):
    v = jnp.dot(ea_ref[...], w_ref[...], preferred_element_type=jnp.float32)
    v = v + b_ref[...]
    e1_ref[...] = v[:, :128]
    eml_ref[...] = v[:, 128:]


def _edense(ea_p, Wc, bc, eb):
    e_pad = ea_p.shape[0]
    grid = e_pad // eb
    return pl.pallas_call(
        _edense_body,
        grid=(grid,),
        in_specs=[
            pl.BlockSpec((eb, ea_p.shape[1]), lambda i: (i, 0)),
            pl.BlockSpec(Wc.shape, lambda i: (0, 0)),
            pl.BlockSpec(bc.shape, lambda i: (0, 0)),
        ],
        out_specs=[
            pl.BlockSpec((eb, 128), lambda i: (i, 0)),
            pl.BlockSpec((eb, 128), lambda i: (i, 0)),
        ],
        out_shape=[
            jax.ShapeDtypeStruct((e_pad, 128), jnp.float32),
            jax.ShapeDtypeStruct((e_pad, 128), jnp.float32),
        ],
    )(ea_p, Wc, bc)


CH = 8


def _sc_agg1(x, src2, dst2, e1, n_acc, ng):
    mesh = plsc.VectorSubcoreMesh(core_axis_name="c", subcore_axis_name="s")
    rows_per_tile = n_acc // NS
    nz = rows_per_tile // G

    @functools.partial(
        pl.kernel,
        mesh=mesh,
        out_type=jax.ShapeDtypeStruct((NC, n_acc, 128), jnp.float32),
        scratch_types=[
            pltpu.VMEM((CH, G), jnp.int32),
            pltpu.VMEM((CH, G), jnp.int32),
            pltpu.VMEM((G, 128), jnp.float32),
            pltpu.VMEM((G, 128), jnp.float32),
            pltpu.VMEM_SHARED((n_acc, 128), jnp.float32),
            pltpu.SemaphoreType.DMA,
        ],
    )
    def body(x_hbm, src_hbm, dst_hbm, e1_hbm, out_hbm,
             src_v, dst_v, xr_v, e1_v, acc, sem):
        c = lax.axis_index("c")
        s = lax.axis_index("s")
        w = s * NC + c

        zeros16 = jnp.zeros((16,), jnp.float32)

        def zb_body(r, carry):
            for cc in range(8):
                e1_v[r, pl.ds(cc * 16, 16)] = zeros16
            return carry

        lax.fori_loop(0, G, zb_body, 0)

        tile_base = s * rows_per_tile

        def zacc_body(k, carry):
            pltpu.sync_copy(e1_v, acc.at[pl.ds(tile_base + k * G, G)])
            return carry

        lax.fori_loop(0, nz, zacc_body, 0)
        plsc.subcore_barrier()

        def block(b, carry):
            row0 = w * ng + b * CH
            pltpu.sync_copy(src_hbm.at[pl.ds(row0, CH)], src_v)
            pltpu.sync_copy(dst_hbm.at[pl.ds(row0, CH)], dst_v)

            def step(j, jc):
                base = (row0 + j) * G
                pltpu.sync_copy(e1_hbm.at[pl.ds(base, G)], e1_v)
                pltpu.async_copy(x_hbm.at[src_v.at[j]], xr_v, sem).wait()

                def rows(r, rc):
                    for cc in range(8):
                        sl = pl.ds(cc * 16, 16)
                        e1_v[r, sl] = jnp.maximum(xr_v[r, sl] + e1_v[r, sl], 0.0)
                    return rc

                lax.fori_loop(0, G, rows, 0)
                pltpu.sync_copy(e1_v, acc.at[dst_v.at[j]], add=True)
                return jc

            lax.fori_loop(0, CH, step, 0)
            return carry

        lax.fori_loop(0, ng // CH, block, 0)
        plsc.subcore_barrier()

        def wout(k, carry):
            r0 = tile_base + k * G
            pltpu.sync_copy(acc.at[pl.ds(r0, G)], out_hbm.at[c, pl.ds(r0, G)])
            return carry

        lax.fori_loop(0, nz, wout, 0)

    return body(x, src2, dst2, e1)


def _sc_agg2(h, src2, dst2, eml, n_acc, ng):
    mesh = plsc.VectorSubcoreMesh(core_axis_name="c", subcore_axis_name="s")
    rows_per_tile = n_acc // NS
    nz = rows_per_tile // G

    @functools.partial(
        pl.kernel,
        mesh=mesh,
        out_type=jax.ShapeDtypeStruct((NC, n_acc, 128), jnp.float32),
        scratch_types=[
            pltpu.VMEM((CH, G), jnp.int32),
            pltpu.VMEM((CH, G), jnp.int32),
            pltpu.VMEM((G, 64), jnp.float32),
            pltpu.VMEM((G, 128), jnp.float32),
            pltpu.VMEM_SHARED((n_acc, 128), jnp.float32),
            pltpu.SemaphoreType.DMA,
        ],
        compiler_params=pltpu.CompilerParams(use_tc_tiling_on_sc=False),
    )
    def body(h_hbm, src_hbm, dst_hbm, eml_hbm, out_hbm,
             src_v, dst_v, hr_v, eml_v, acc, sem):
        c = lax.axis_index("c")
        s = lax.axis_index("s")
        w = s * NC + c

        zeros16 = jnp.zeros((16,), jnp.float32)

        def zb_body(r, carry):
            for cc in range(8):
                eml_v[r, pl.ds(cc * 16, 16)] = zeros16
            return carry

        lax.fori_loop(0, G, zb_body, 0)

        tile_base = s * rows_per_tile

        def zacc_body(k, carry):
            pltpu.sync_copy(eml_v, acc.at[pl.ds(tile_base + k * G, G)])
            return carry

        lax.fori_loop(0, nz, zacc_body, 0)
        plsc.subcore_barrier()

        def block(b, carry):
            row0 = w * ng + b * CH
            pltpu.sync_copy(src_hbm.at[pl.ds(row0, CH)], src_v)
            pltpu.sync_copy(dst_hbm.at[pl.ds(row0, CH)], dst_v)

            def step(j, jc):
                base = (row0 + j) * G
                pltpu.sync_copy(eml_hbm.at[pl.ds(base, G)], eml_v)
                pltpu.async_copy(h_hbm.at[src_v.at[j]], hr_v, sem).wait()

                def rows(r, rc):
                    for cc in range(4):
                        sl = pl.ds(cc * 16, 16)
                        sl2 = pl.ds(64 + cc * 16, 16)
                        hv = hr_v[r, sl]
                        eml_v[r, sl] = jnp.maximum(hv + eml_v[r, sl], 0.0)
                        eml_v[r, sl2] = jnp.maximum(hv + eml_v[r, sl2], 0.0)
                    return rc

                lax.fori_loop(0, G, rows, 0)
                pltpu.sync_copy(eml_v, acc.at[dst_v.at[j]], add=True)
                return jc

            lax.fori_loop(0, CH, step, 0)
            return carry

        lax.fori_loop(0, ng // CH, block, 0)
        plsc.subcore_barrier()

        def wout(k, carry):
            r0 = tile_base + k * G
            pltpu.sync_copy(acc.at[pl.ds(r0, G)], out_hbm.at[c, pl.ds(r0, G)])
            return carry

        lax.fori_loop(0, nz, wout, 0)

    return body(h, src2, dst2, eml)


def _node1_body(x_ref, p_ref, wa_ref, ba_ref, wb_ref, bb_ref, h_ref):
    h1 = x_ref[...] + p_ref[0] + p_ref[1]
    t = jnp.maximum(jnp.dot(h1, wa_ref[...], preferred_element_type=jnp.float32)
                    + ba_ref[...], 0.0)
    g = jnp.dot(t, wb_ref[...], preferred_element_type=jnp.float32) + bb_ref[...]
    h_ref[...] = jnp.maximum(g, 0.0)


def _node1(x, P1, W1a, b1a, W1b, b1b, nb):
    n, d = x.shape
    hdim = W1a.shape[1]
    grid = n // nb
    return pl.pallas_call(
        _node1_body,
        grid=(grid,),
        in_specs=[
            pl.BlockSpec((nb, d), lambda i: (i, 0)),
            pl.BlockSpec((NC, nb, d), lambda i: (0, i, 0)),
            pl.BlockSpec(W1a.shape, lambda i: (0, 0)),
            pl.BlockSpec(b1a.shape, lambda i: (0, 0)),
            pl.BlockSpec(W1b.shape, lambda i: (0, 0)),
            pl.BlockSpec(b1b.shape, lambda i: (0, 0)),
        ],
        out_specs=pl.BlockSpec((nb, hdim), lambda i: (i, 0)),
        out_shape=jax.ShapeDtypeStruct((n, hdim), jnp.float32),
    )(x, P1, W1a, b1a, W1b, b1b)


def _node2_body(h_ref, p_ref, wm1_ref, bm1_ref, wm2_ref, bm2_ref,
                wl1_ref, bl1_ref, wl2_ref, bl2_ref, mu_ref, ls_ref):
    hb = h_ref[...]
    hm = hb + p_ref[0, :, :64] + p_ref[1, :, :64]
    hl = hb + p_ref[0, :, 64:] + p_ref[1, :, 64:]
    tm = jnp.maximum(jnp.dot(hm, wm1_ref[...], preferred_element_type=jnp.float32)
                     + bm1_ref[...], 0.0)
    mu_ref[...] = jnp.dot(tm, wm2_ref[...], preferred_element_type=jnp.float32) + bm2_ref[...]
    tl = jnp.maximum(jnp.dot(hl, wl1_ref[...], preferred_element_type=jnp.float32)
                     + bl1_ref[...], 0.0)
    ls = jnp.dot(tl, wl2_ref[...], preferred_element_type=jnp.float32) + bl2_ref[...]
    ls_ref[...] = jnp.clip(ls, -10.0, 10.0)


def _node2(h, P2, Wm1, bm1, Wm2, bm2, Wl1, bl1, Wl2, bl2, nb):
    n, hdim = h.shape
    ldim = Wm2.shape[1]
    grid = n // nb
    wspec = lambda shp: pl.BlockSpec(shp, lambda i: (0, 0))
    return pl.pallas_call(
        _node2_body,
        grid=(grid,),
        in_specs=[
            pl.BlockSpec((nb, hdim), lambda i: (i, 0)),
            pl.BlockSpec((NC, nb, 128), lambda i: (0, i, 0)),
            wspec(Wm1.shape), wspec(bm1.shape), wspec(Wm2.shape), wspec(bm2.shape),
            wspec(Wl1.shape), wspec(bl1.shape), wspec(Wl2.shape), wspec(bl2.shape),
        ],
        out_specs=[
            pl.BlockSpec((nb, ldim), lambda i: (i, 0)),
            pl.BlockSpec((nb, ldim), lambda i: (i, 0)),
        ],
        out_shape=[
            jax.ShapeDtypeStruct((n, ldim), jnp.float32),
            jax.ShapeDtypeStruct((n, ldim), jnp.float32),
        ],
    )(h, P2, Wm1, bm1, Wm2, bm2, Wl1, bl1, Wl2, bl2)


@jax.jit
def kernel(x, edge_index, edge_attr, We1, be1, W1a, b1a, W1b, b1b,
           Wem, bem, Wm1, bm1, Wm2, bm2, Wel, bel, Wl1, bl1, Wl2, bl2):
    n, d = x.shape
    e = edge_index.shape[1]
    ed = edge_attr.shape[1]

    ng = _ceil_to((e + NW * G - 1) // (NW * G), 2)
    e_pad = NW * ng * G
    n_acc = _ceil_to(n + 1, NS * G)

    pad = e_pad - e
    src = edge_index[0].astype(jnp.int32)
    dst = edge_index[1].astype(jnp.int32)
    src2 = jnp.concatenate([src, jnp.zeros((pad,), jnp.int32)]).reshape(e_pad // G, G)
    dst2 = jnp.concatenate([dst, jnp.full((pad,), n, jnp.int32)]).reshape(e_pad // G, G)
    ea_p = jnp.concatenate([edge_attr, jnp.zeros((pad, ed), jnp.float32)])

    Wc = jnp.concatenate([We1, Wem, Wel], axis=1)
    bc = jnp.concatenate([be1, bem, bel]).reshape(1, 256)

    e1, eml = _edense(ea_p, Wc, bc, eb=8192)
    P1 = _sc_agg1(x, src2, dst2, e1, n_acc, ng)
    P1 = P1[:, :n, :]
    h = _node1(x, P1, W1a, b1a.reshape(1, -1), W1b, b1b.reshape(1, -1), nb=2000)
    P2 = _sc_agg2(h, src2, dst2, eml, n_acc, ng)
    P2 = P2[:, :n, :]
    mu, logstd = _node2(h, P2, Wm1, bm1.reshape(1, -1), Wm2, bm2.reshape(1, -1),
                        Wl1, bl1.reshape(1, -1), Wl2, bl2.reshape(1, -1), nb=2000)
    return (mu, logstd)


# speedup vs baseline: 2.4376x; 1.0987x over previous
import functools
import jax
import jax.numpy as jnp
from jax import lax
from jax.experimental import pallas as pl
from jax.experimental.pallas import tpu as pltpu
from jax.experimental.pallas import tpu_sc as plsc

NC = 2
NS = 16
NW = NC * NS
G = 64
CH = 16


def _ceil_to(a, b):
    return (a + b - 1) // b * b


def _edense_body(ea_ref, w_ref, b_ref, e1_ref, eml_ref):
    v = jnp.dot(ea_ref[...], w_ref[...], preferred_element_type=jnp.float32)
    v = v + b_ref[...]
    e1_ref[...] = v[:, :128]
    eml_ref[...] = v[:, 128:]


def _edense(ea_p, Wc, bc, eb):
    e_pad = ea_p.shape[0]
    grid = e_pad // eb
    return pl.pallas_call(
        _edense_body,
        grid=(grid,),
        in_specs=[
            pl.BlockSpec((eb, ea_p.shape[1]), lambda i: (i, 0)),
            pl.BlockSpec(Wc.shape, lambda i: (0, 0)),
            pl.BlockSpec(bc.shape, lambda i: (0, 0)),
        ],
        out_specs=[
            pl.BlockSpec((eb, 128), lambda i: (i, 0)),
            pl.BlockSpec((eb, 128), lambda i: (i, 0)),
        ],
        out_shape=[
            jax.ShapeDtypeStruct((e_pad, 128), jnp.float32),
            jax.ShapeDtypeStruct((e_pad, 128), jnp.float32),
        ],
    )(ea_p, Wc, bc)


def _make_agg(table_w, emb_w, compute_rows, n_acc, ng, tc_tiling):
    mesh = plsc.VectorSubcoreMesh(core_axis_name="c", subcore_axis_name="s")
    rows_per_tile = n_acc // NS
    nz = rows_per_tile // G
    nsteps = ng
    cp = pltpu.CompilerParams(use_tc_tiling_on_sc=tc_tiling)

    if True:
        @functools.partial(
            pl.kernel,
            mesh=mesh,
            out_type=jax.ShapeDtypeStruct((NC, n_acc, emb_w), jnp.float32),
            compiler_params=cp,
            scratch_types=[
                pltpu.VMEM((CH, G), jnp.int32),
                pltpu.VMEM((CH, G), jnp.int32),
                pltpu.VMEM((2, G, table_w), jnp.float32),
                pltpu.VMEM((2, G, emb_w), jnp.float32),
                pltpu.SemaphoreType.DMA,
                pltpu.SemaphoreType.DMA,
                pltpu.SemaphoreType.DMA,
                pltpu.SemaphoreType.DMA,
                pltpu.SemaphoreType.DMA,
                pltpu.SemaphoreType.DMA,
                pltpu.VMEM_SHARED((n_acc, emb_w), jnp.float32),
            ],
        )
        def body(tbl_hbm, src_hbm, dst_hbm, emb_hbm, out_hbm,
                 src_v, dst_v, tbl_v, emb_v, gsem0, gsem1, esem0, esem1,
                 ssem0, ssem1, acc):
            c = lax.axis_index("c")
            s = lax.axis_index("s")
            w = s * NC + c
            gsem = (gsem0, gsem1)
            esem = (esem0, esem1)
            ssem = (ssem0, ssem1)

            zeros16 = jnp.zeros((16,), jnp.float32)

            def zb_body(r, carry):
                for cc in range(emb_w // 16):
                    emb_v[0, r, pl.ds(cc * 16, 16)] = zeros16
                return carry

            lax.fori_loop(0, G, zb_body, 0)
            tile_base = s * rows_per_tile

            def zacc_body(k, carry):
                pltpu.sync_copy(emb_v.at[0], acc.at[pl.ds(tile_base + k * G, G)])
                return carry

            lax.fori_loop(0, nz, zacc_body, 0)
            plsc.subcore_barrier()

            def issue_loads(j, b):
                blk = j // CH
                jj = j - blk * CH
                pltpu.async_copy(
                    emb_hbm.at[pl.ds((w * ng + j) * G, G)], emb_v.at[b],
                    esem[b])
                pltpu.async_copy(tbl_hbm.at[src_v.at[jj]], tbl_v.at[b],
                                 gsem[b])

            def wait_loads(b):
                pltpu.make_async_copy(
                    emb_hbm.at[pl.ds(0, G)], emb_v.at[b], esem[b]).wait()
                pltpu.make_async_copy(
                    tbl_hbm.at[src_v.at[0]], tbl_v.at[b], gsem[b]).wait()

            def issue_scatter(j, b):
                blk = j // CH
                jj = j - blk * CH
                pltpu.async_copy(emb_v.at[b], acc.at[dst_v.at[jj]], ssem[b],
                                 add=True)

            def wait_scatter(b):
                pltpu.make_async_copy(
                    emb_v.at[b], acc.at[dst_v.at[0]], ssem[b]).wait()

            def load_idx(blk):
                pltpu.sync_copy(src_hbm.at[pl.ds(w * ng + blk * CH, CH)], src_v)
                pltpu.sync_copy(dst_hbm.at[pl.ds(w * ng + blk * CH, CH)], dst_v)

            load_idx(0)
            issue_loads(0, 0)
            assert nsteps % 2 == 0 and CH % 2 == 0

            def step2(jj, carry):
                for b in range(2):
                    j = jj * 2 + b
                    wait_loads(b)

                    def rows(r, rc, b=b):
                        compute_rows(tbl_v, emb_v, b, r)
                        return rc

                    lax.fori_loop(0, G, rows, 0)

                    if b == 0:
                        @pl.when(jnp.logical_and(jj >= 1,
                                                 lax.rem(jj * 2, CH) != 0))
                        def _():
                            wait_scatter(1)
                    else:
                        wait_scatter(0)

                    issue_scatter(j, b)

                    nxt = j + 1
                    if b == 0:
                        issue_loads(nxt, 1)
                    else:
                        @pl.when(jj + 1 < nsteps // 2)
                        def _():
                            @pl.when(lax.rem(nxt, CH) == 0)
                            def _():
                                wait_scatter(1)
                                load_idx(nxt // CH)

                            issue_loads(nxt, 0)
                return carry

            lax.fori_loop(0, nsteps // 2, step2, 0)
            wait_scatter(1)
            plsc.subcore_barrier()

            def wout(k, carry):
                r0 = tile_base + k * G
                pltpu.sync_copy(acc.at[pl.ds(r0, G)], out_hbm.at[c, pl.ds(r0, G)])
                return carry

            lax.fori_loop(0, nz, wout, 0)

    return body


def _rows_l1(tbl_v, emb_v, b, r):
    for cc in range(8):
        sl = pl.ds(cc * 16, 16)
        emb_v[b, r, sl] = jnp.maximum(tbl_v[b, r, sl] + emb_v[b, r, sl], 0.0)


def _rows_l23(tbl_v, emb_v, b, r):
    for cc in range(4):
        sl = pl.ds(cc * 16, 16)
        sl2 = pl.ds(64 + cc * 16, 16)
        hv = tbl_v[b, r, sl]
        emb_v[b, r, sl] = jnp.maximum(hv + emb_v[b, r, sl], 0.0)
        emb_v[b, r, sl2] = jnp.maximum(hv + emb_v[b, r, sl2], 0.0)


def _sc_agg1(x, src2, dst2, e1, n_acc, ng):
    body = _make_agg(128, 128, _rows_l1, n_acc, ng, tc_tiling=True)
    return body(x, src2, dst2, e1)


def _sc_agg2(h, src2, dst2, eml, n_acc, ng):
    body = _make_agg(64, 128, _rows_l23, n_acc, ng, tc_tiling=False)
    return body(h, src2, dst2, eml)


def _node1_body(x_ref, p_ref, wa_ref, ba_ref, wb_ref, bb_ref, h_ref):
    h1 = x_ref[...] + p_ref[0] + p_ref[1]
    t = jnp.maximum(jnp.dot(h1, wa_ref[...], preferred_element_type=jnp.float32)
                    + ba_ref[...], 0.0)
    g = jnp.dot(t, wb_ref[...], preferred_element_type=jnp.float32) + bb_ref[...]
    h_ref[...] = jnp.maximum(g, 0.0)


def _node1(x, P1, W1a, b1a, W1b, b1b, nb):
    n, d = x.shape
    hdim = W1a.shape[1]
    grid = n // nb
    return pl.pallas_call(
        _node1_body,
        grid=(grid,),
        in_specs=[
            pl.BlockSpec((nb, d), lambda i: (i, 0)),
            pl.BlockSpec((NC, nb, d), lambda i: (0, i, 0)),
            pl.BlockSpec(W1a.shape, lambda i: (0, 0)),
            pl.BlockSpec(b1a.shape, lambda i: (0, 0)),
            pl.BlockSpec(W1b.shape, lambda i: (0, 0)),
            pl.BlockSpec(b1b.shape, lambda i: (0, 0)),
        ],
        out_specs=pl.BlockSpec((nb, hdim), lambda i: (i, 0)),
        out_shape=jax.ShapeDtypeStruct((n, hdim), jnp.float32),
    )(x, P1, W1a, b1a, W1b, b1b)


def _node2_body(h_ref, p_ref, wm1_ref, bm1_ref, wm2_ref, bm2_ref,
                wl1_ref, bl1_ref, wl2_ref, bl2_ref, mu_ref, ls_ref):
    hb = h_ref[...]
    hm = hb + p_ref[0, :, :64] + p_ref[1, :, :64]
    hl = hb + p_ref[0, :, 64:] + p_ref[1, :, 64:]
    tm = jnp.maximum(jnp.dot(hm, wm1_ref[...], preferred_element_type=jnp.float32)
                     + bm1_ref[...], 0.0)
    mu_ref[...] = jnp.dot(tm, wm2_ref[...], preferred_element_type=jnp.float32) + bm2_ref[...]
    tl = jnp.maximum(jnp.dot(hl, wl1_ref[...], preferred_element_type=jnp.float32)
                     + bl1_ref[...], 0.0)
    ls = jnp.dot(tl, wl2_ref[...], preferred_element_type=jnp.float32) + bl2_ref[...]
    ls_ref[...] = jnp.clip(ls, -10.0, 10.0)


def _node2(h, P2, Wm1, bm1, Wm2, bm2, Wl1, bl1, Wl2, bl2, nb):
    n, hdim = h.shape
    ldim = Wm2.shape[1]
    grid = n // nb
    wspec = lambda shp: pl.BlockSpec(shp, lambda i: (0, 0))
    return pl.pallas_call(
        _node2_body,
        grid=(grid,),
        in_specs=[
            pl.BlockSpec((nb, hdim), lambda i: (i, 0)),
            pl.BlockSpec((NC, nb, 128), lambda i: (0, i, 0)),
            wspec(Wm1.shape), wspec(bm1.shape), wspec(Wm2.shape), wspec(bm2.shape),
            wspec(Wl1.shape), wspec(bl1.shape), wspec(Wl2.shape), wspec(bl2.shape),
        ],
        out_specs=[
            pl.BlockSpec((nb, ldim), lambda i: (i, 0)),
            pl.BlockSpec((nb, ldim), lambda i: (i, 0)),
        ],
        out_shape=[
            jax.ShapeDtypeStruct((n, ldim), jnp.float32),
            jax.ShapeDtypeStruct((n, ldim), jnp.float32),
        ],
    )(h, P2, Wm1, bm1, Wm2, bm2, Wl1, bl1, Wl2, bl2)


@jax.jit
def kernel(x, edge_index, edge_attr, We1, be1, W1a, b1a, W1b, b1b,
           Wem, bem, Wm1, bm1, Wm2, bm2, Wel, bel, Wl1, bl1, Wl2, bl2):
    n, d = x.shape
    e = edge_index.shape[1]
    ed = edge_attr.shape[1]

    ng = _ceil_to((e + NW * G - 1) // (NW * G), 8)
    e_pad = NW * ng * G
    n_acc = _ceil_to(n + 1, NS * G)

    pad = e_pad - e
    src = edge_index[0].astype(jnp.int32)
    dst = edge_index[1].astype(jnp.int32)
    src2 = jnp.concatenate([src, jnp.zeros((pad,), jnp.int32)]).reshape(e_pad // G, G)
    dst2 = jnp.concatenate([dst, jnp.full((pad,), n, jnp.int32)]).reshape(e_pad // G, G)
    ea_p = jnp.concatenate([edge_attr, jnp.zeros((pad, ed), jnp.float32)])

    Wc = jnp.concatenate([We1, Wem, Wel], axis=1)
    bc = jnp.concatenate([be1, bem, bel]).reshape(1, 256)

    e1, eml = _edense(ea_p, Wc, bc, eb=8192)
    P1 = _sc_agg1(x, src2, dst2, e1, n_acc, ng)
    P1 = P1[:, :n, :]
    h = _node1(x, P1, W1a, b1a.reshape(1, -1), W1b, b1b.reshape(1, -1), nb=2000)
    P2 = _sc_agg2(h, src2, dst2, eml, n_acc, ng)
    P2 = P2[:, :n, :]
    mu, logstd = _node2(h, P2, Wm1, bm1.reshape(1, -1), Wm2, bm2.reshape(1, -1),
                        Wl1, bl1.reshape(1, -1), Wl2, bl2.reshape(1, -1), nb=2000)
    return (mu, logstd)
